# baseline (speedup 1.0000x reference)
import functools

import jax
import jax.numpy as jnp
from jax.experimental import pallas as pl
from jax.experimental.pallas import tpu as pltpu
from jax.experimental.pallas import tpu_sc as plsc

MODEL_DIM = 4096
PROMPT_DIM = 64
NUM_EXPERTS = 64
T_TILE = 1024
T = 32768

NW = 32
ROWS_PER_W = T // NW
RB = 128
NB = ROWS_PER_W // RB


def _logits_kernel(x_ref, p_ref, wm_ref, wp_ref, b_ref, lg_ref):
    logits = jax.lax.dot_general(
        x_ref[...], wm_ref[...], (((1,), (1,)), ((), ())),
        precision=jax.lax.Precision.DEFAULT)
    logits = logits + jax.lax.dot_general(
        p_ref[...], wp_ref[...], (((1,), (1,)), ((), ())),
        precision=jax.lax.Precision.DEFAULT)
    lg_ref[...] = logits + b_ref[...]


def _tc_logits(input, prompt, Wm, Wp, b2):
    grid = (T // T_TILE,)
    return pl.pallas_call(
        _logits_kernel,
        grid=grid,
        in_specs=[
            pl.BlockSpec((T_TILE, MODEL_DIM), lambda i: (i, 0)),
            pl.BlockSpec((T_TILE, PROMPT_DIM), lambda i: (i, 0)),
            pl.BlockSpec((NUM_EXPERTS, MODEL_DIM), lambda i: (0, 0)),
            pl.BlockSpec((NUM_EXPERTS, PROMPT_DIM), lambda i: (0, 0)),
            pl.BlockSpec((1, NUM_EXPERTS), lambda i: (0, 0)),
        ],
        out_specs=pl.BlockSpec((T_TILE, NUM_EXPERTS), lambda i: (i, 0)),
        out_shape=jax.ShapeDtypeStruct((T, NUM_EXPERTS), jnp.float32),
        compiler_params=pltpu.CompilerParams(
            dimension_semantics=("arbitrary",)),
    )(input, prompt, Wm, Wp, b2)


def _sc_gate_body(lg_hbm, m0_hbm, m1_hbm, g0_hbm, g1_hbm,
                  lbuf, m0buf, m1buf, g0buf, g1buf, i1buf, i2buf):
    c = jax.lax.axis_index("c")
    s = jax.lax.axis_index("s")
    wid = s * 2 + c
    base = wid * ROWS_PER_W
    iota = jax.lax.broadcasted_iota(jnp.int32, (16,), 0)
    neg = jnp.full((16,), -jnp.inf, jnp.float32)
    zi = jnp.zeros((16,), jnp.int32)
    ones = jnp.ones((16,), jnp.int32)

    def zr(r, carry):
        for cc in range(4):
            m0buf[r, pl.ds(16 * cc, 16)] = zi
            m1buf[r, pl.ds(16 * cc, 16)] = zi
        return carry

    jax.lax.fori_loop(0, RB, zr, 0)

    def block(bi, carry):
        rbase = base + bi * RB
        pltpu.sync_copy(lg_hbm.at[pl.ds(rbase, RB)], lbuf)

        def grp(gi, carry3):
            rows = gi * 16 + iota

            def estep(e, acc):
                M1, I1, M2, I2 = acc
                ev = jnp.full((16,), e, jnp.int32)
                v = plsc.load_gather(lbuf, [rows, ev])
                gt1 = v > M1
                gt2 = v > M2
                M2n = jnp.where(gt1, M1, jnp.where(gt2, v, M2))
                I2n = jnp.where(gt1, I1, jnp.where(gt2, ev, I2))
                M1n = jnp.where(gt1, v, M1)
                I1n = jnp.where(gt1, ev, I1)
                return (M1n, I1n, M2n, I2n)

            M1, I1, M2, I2 = jax.lax.fori_loop(
                0, NUM_EXPERTS, estep, (neg, zi, neg, zi))
            plsc.store_scatter(m0buf, [rows, I1], ones)
            plsc.store_scatter(m1buf, [rows, I2], ones)
            i1buf[pl.ds(gi * 16, 16)] = I1
            i2buf[pl.ds(gi * 16, 16)] = I2
            eg = jnp.exp(M2 - M1)
            G0 = 1.0 / (1.0 + eg)
            g0buf[pl.ds(gi * 16, 16)] = G0
            g1buf[pl.ds(gi * 16, 16)] = 1.0 - G0
            return carry3

        jax.lax.fori_loop(0, RB // 16, grp, 0)

        pltpu.sync_copy(m0buf, m0_hbm.at[pl.ds(rbase, RB)])
        pltpu.sync_copy(m1buf, m1_hbm.at[pl.ds(rbase, RB)])
        pltpu.sync_copy(g0buf, g0_hbm.at[pl.ds(rbase, RB)])
        pltpu.sync_copy(g1buf, g1_hbm.at[pl.ds(rbase, RB)])

        def clear(gi, carry4):
            rows = gi * 16 + iota
            plsc.store_scatter(m0buf, [rows, i1buf[pl.ds(gi * 16, 16)]], zi)
            plsc.store_scatter(m1buf, [rows, i2buf[pl.ds(gi * 16, 16)]], zi)
            return carry4

        jax.lax.fori_loop(0, RB // 16, clear, 0)
        return carry

    jax.lax.fori_loop(0, NB, block, 0)


_sc_gate = functools.partial(
    pl.kernel,
    mesh=plsc.VectorSubcoreMesh(core_axis_name="c", subcore_axis_name="s"),
    compiler_params=pltpu.CompilerParams(needs_layout_passes=False),
    out_type=[
        jax.ShapeDtypeStruct((T, NUM_EXPERTS), jnp.int32),
        jax.ShapeDtypeStruct((T, NUM_EXPERTS), jnp.int32),
        jax.ShapeDtypeStruct((T,), jnp.float32),
        jax.ShapeDtypeStruct((T,), jnp.float32),
    ],
    scratch_types=[
        pltpu.VMEM((RB, NUM_EXPERTS), jnp.float32),
        pltpu.VMEM((RB, NUM_EXPERTS), jnp.int32),
        pltpu.VMEM((RB, NUM_EXPERTS), jnp.int32),
        pltpu.VMEM((RB,), jnp.float32),
        pltpu.VMEM((RB,), jnp.float32),
        pltpu.VMEM((RB,), jnp.int32),
        pltpu.VMEM((RB,), jnp.int32),
    ],
)(_sc_gate_body)


def kernel(input, prompt, W, b):
    Wm = W[:, :MODEL_DIM]
    Wp = W[:, MODEL_DIM:]
    b2 = b.reshape(1, NUM_EXPERTS)
    logits = _tc_logits(input, prompt, Wm, Wp, b2)
    m0, m1, g0, g1 = _sc_gate(logits)
    return m0, m1, g0, g1


# speedup vs baseline: 1.2383x; 1.2383x over previous
import jax
import jax.numpy as jnp
from jax.experimental import pallas as pl
from jax.experimental.pallas import tpu as pltpu

MODEL_DIM = 4096
PROMPT_DIM = 64
NUM_EXPERTS = 64
T_TILE = 1024


def _gate_kernel(x_ref, p_ref, wm_ref, wp_ref, b_ref,
                 m0_ref, m1_ref, g0_ref, g1_ref):
    x = x_ref[...]
    p = p_ref[...]
    logits = jax.lax.dot_general(
        x, wm_ref[...], (((1,), (1,)), ((), ())),
        precision=jax.lax.Precision.DEFAULT)
    logits = logits + jax.lax.dot_general(
        p, wp_ref[...], (((1,), (1,)), ((), ())),
        precision=jax.lax.Precision.DEFAULT)
    logits = logits + b_ref[...]

    rows = logits.shape[0]
    cols = jax.lax.broadcasted_iota(jnp.int32, (rows, NUM_EXPERTS), 1)
    l1 = jnp.max(logits, axis=1, keepdims=True)
    i1 = jnp.min(jnp.where(logits == l1, cols, NUM_EXPERTS), axis=1)
    onehot1 = cols == i1[:, None]
    masked = jnp.where(onehot1, -jnp.inf, logits)
    l2 = jnp.max(masked, axis=1, keepdims=True)
    i2 = jnp.min(jnp.where(masked == l2, cols, NUM_EXPERTS), axis=1)
    onehot2 = cols == i2[:, None]

    e = jnp.exp(l2 - l1)
    g0 = 1.0 / (1.0 + e)
    m0_ref[...] = onehot1.astype(jnp.int32)
    m1_ref[...] = onehot2.astype(jnp.int32)
    g0_ref[...] = g0
    g1_ref[...] = 1.0 - g0


def kernel(input, prompt, W, b):
    T = input.shape[0]
    Wm = W[:, :MODEL_DIM]
    Wp = W[:, MODEL_DIM:]
    b2 = b.reshape(1, NUM_EXPERTS)
    grid = (T // T_TILE,)
    m0, m1, g0, g1 = pl.pallas_call(
        _gate_kernel,
        grid=grid,
        in_specs=[
            pl.BlockSpec((T_TILE, MODEL_DIM), lambda i: (i, 0)),
            pl.BlockSpec((T_TILE, PROMPT_DIM), lambda i: (i, 0)),
            pl.BlockSpec((NUM_EXPERTS, MODEL_DIM), lambda i: (0, 0)),
            pl.BlockSpec((NUM_EXPERTS, PROMPT_DIM), lambda i: (0, 0)),
            pl.BlockSpec((1, NUM_EXPERTS), lambda i: (0, 0)),
        ],
        out_specs=[
            pl.BlockSpec((T_TILE, NUM_EXPERTS), lambda i: (i, 0)),
            pl.BlockSpec((T_TILE, NUM_EXPERTS), lambda i: (i, 0)),
            pl.BlockSpec((T_TILE, 1), lambda i: (i, 0)),
            pl.BlockSpec((T_TILE, 1), lambda i: (i, 0)),
        ],
        out_shape=[
            jax.ShapeDtypeStruct((T, NUM_EXPERTS), jnp.int32),
            jax.ShapeDtypeStruct((T, NUM_EXPERTS), jnp.int32),
            jax.ShapeDtypeStruct((T, 1), jnp.float32),
            jax.ShapeDtypeStruct((T, 1), jnp.float32),
        ],
        compiler_params=pltpu.CompilerParams(
            dimension_semantics=("arbitrary",)),
    )(input, prompt, Wm, Wp, b2)
    return m0, m1, g0.reshape(T), g1.reshape(T)
